# baseline (speedup 1.0000x reference)
import functools

import jax
import jax.numpy as jnp
from jax import lax
from jax.experimental import pallas as pl
from jax.experimental.pallas import tpu as pltpu
from jax.experimental.pallas import tpu_sc as plsc

L = 16
CHUNK = 32
GROUP = 8


def _fm_body(F, V, D, rows_per_worker, n_chunks,
             utab, itab, uidx, iidx, wb_hbm, out,
             uraw, iraw, ulist, ilist, uoff, ioff, ubuf, ibuf,
             score_v, wb_v, sem):
    nc = 2
    wid = lax.axis_index("s") * nc + lax.axis_index("c")
    base0 = wid * rows_per_worker

    pltpu.sync_copy(wb_hbm, wb_v)
    wb = wb_v[pl.ds(0, L)]
    w = wb[0]
    bb = wb[1]

    lane = lax.iota(jnp.int32, L)

    def chunk_body(c, _):
        base = base0 + c * CHUNK
        pltpu.sync_copy(uidx.at[pl.ds(base, CHUNK)], uraw)
        pltpu.sync_copy(iidx.at[pl.ds(base, CHUNK)], iraw)

        def tr_body(g, _):
            rvec = g * L + lane
            for f in range(F):
                fvec = jnp.full((L,), f, jnp.int32)
                uv = plsc.load_gather(uraw, [rvec, fvec]) + f * V
                iv = plsc.load_gather(iraw, [rvec, fvec]) + f * V
                ulist[f, pl.ds(g * L, L)] = lax.shift_right_logical(uv, 3)
                ilist[f, pl.ds(g * L, L)] = lax.shift_right_logical(iv, 3)
                uoff[f, pl.ds(g * L, L)] = lax.bitwise_and(uv, 7) * D
                ioff[f, pl.ds(g * L, L)] = lax.bitwise_and(iv, 7) * D
            return 0
        lax.fori_loop(0, CHUNK // L, tr_body, 0)

        copies = []
        for f in range(F):
            copies.append(pltpu.async_copy(utab.at[ulist.at[f]],
                                           ubuf.at[f], sem))
            copies.append(pltpu.async_copy(itab.at[ilist.at[f]],
                                           ibuf.at[f], sem))
        for cp in copies:
            cp.wait()

        def group_body(g, _):
            svec = jnp.zeros((L,), jnp.float32)
            uo = [uoff[f, pl.ds(g * L, L)] for f in range(F)]
            io = [ioff[f, pl.ds(g * L, L)] for f in range(F)]
            for r16 in range(L):
                j = g * L + r16
                jj = jnp.full((L,), 0, jnp.int32) + j
                u = jnp.zeros((L,), jnp.float32)
                v = jnp.zeros((L,), jnp.float32)
                for f in range(F):
                    ff = jnp.full((L,), f, jnp.int32)
                    u = u + plsc.load_gather(ubuf, [ff, jj, uo[f][r16] + lane])
                    v = v + plsc.load_gather(ibuf, [ff, jj, io[f][r16] + lane])
                s = jnp.sum(u * v)
                svec = jnp.where(lane == r16, s, svec)
            z = svec * w + bb
            score_v[pl.ds(g * L, L)] = 1.0 / (1.0 + jnp.exp(-z))
            return 0
        lax.fori_loop(0, CHUNK // L, group_body, 0)

        pltpu.sync_copy(score_v, out.at[pl.ds(base, CHUNK)])
        return 0

    lax.fori_loop(0, n_chunks, chunk_body, 0)


def kernel(user_idx, item_idx, user_tables, item_tables, W, b):
    B, F = user_idx.shape
    _, V, D = user_tables.shape
    n_workers = 32
    rows_per_worker = B // n_workers
    n_chunks = rows_per_worker // CHUNK
    n_groups = F * V * D // (GROUP * D)

    utab = user_tables.reshape(n_groups, GROUP * D)
    itab = item_tables.reshape(n_groups, GROUP * D)
    wb = jnp.zeros((128,), jnp.float32).at[0].set(W[0, 0]).at[1].set(b[0])

    mesh = plsc.VectorSubcoreMesh(core_axis_name="c", subcore_axis_name="s")
    fm = pl.kernel(
        functools.partial(_fm_body, F, V, D, rows_per_worker, n_chunks),
        out_type=jax.ShapeDtypeStruct((B,), jnp.float32),
        mesh=mesh,
        compiler_params=pltpu.CompilerParams(needs_layout_passes=False,
                                             use_tc_tiling_on_sc=True),
        scratch_types=[
            pltpu.VMEM((CHUNK, F), jnp.int32),
            pltpu.VMEM((CHUNK, F), jnp.int32),
            pltpu.VMEM((F, CHUNK), jnp.int32),
            pltpu.VMEM((F, CHUNK), jnp.int32),
            pltpu.VMEM((F, CHUNK), jnp.int32),
            pltpu.VMEM((F, CHUNK), jnp.int32),
            pltpu.VMEM((F, CHUNK, GROUP * D), jnp.float32),
            pltpu.VMEM((F, CHUNK, GROUP * D), jnp.float32),
            pltpu.VMEM((CHUNK,), jnp.float32),
            pltpu.VMEM((128,), jnp.float32),
            pltpu.SemaphoreType.DMA,
        ],
    )
    out = fm(utab, itab, user_idx, item_idx, wb)
    return out.reshape(B, 1)


# speedup vs baseline: 1.0721x; 1.0721x over previous
import functools

import jax
import jax.numpy as jnp
from jax import lax
from jax.experimental import pallas as pl
from jax.experimental.pallas import tpu as pltpu
from jax.experimental.pallas import tpu_sc as plsc

L = 16
CHUNK = 128


def _fm_body(F, V, D, rows_per_worker, n_chunks,
             utab, itab, uidx, iidx, wb_hbm, out,
             uraw, iraw, ulist, ilist, ubuf, ibuf, score_v, wb_v, sem):
    nc = 2
    wid = lax.axis_index("s") * nc + lax.axis_index("c")
    base0 = wid * rows_per_worker

    pltpu.sync_copy(wb_hbm, wb_v)
    wb = wb_v[:]
    w = wb[0]
    bb = wb[1]

    lane = lax.iota(jnp.int32, L)

    def chunk_body(c, _):
        base = base0 + c * CHUNK
        pltpu.sync_copy(uidx.at[pl.ds(base, CHUNK)], uraw)
        pltpu.sync_copy(iidx.at[pl.ds(base, CHUNK)], iraw)

        def tr_body(g, _):
            rvec = g * L + lane
            for f in range(F):
                fvec = jnp.full((L,), f, jnp.int32)
                uv = plsc.load_gather(uraw, [rvec, fvec])
                iv = plsc.load_gather(iraw, [rvec, fvec])
                ulist[f, pl.ds(g * L, L)] = uv
                ilist[f, pl.ds(g * L, L)] = iv
            return 0
        lax.fori_loop(0, CHUNK // L, tr_body, 0)

        copies = []
        for f in range(F):
            copies.append(pltpu.async_copy(utab.at[f].at[ulist.at[f]],
                                           ubuf.at[f], sem))
            copies.append(pltpu.async_copy(itab.at[f].at[ilist.at[f]],
                                           ibuf.at[f], sem))
        for cp in copies:
            cp.wait()

        def group_body(g, _):
            svec = jnp.zeros((L,), jnp.float32)
            for r16 in range(L):
                r = g * L + r16
                u = ubuf[0, r]
                v = ibuf[0, r]
                for f in range(1, F):
                    u = u + ubuf[f, r]
                    v = v + ibuf[f, r]
                s = jnp.sum(u * v)
                svec = jnp.where(lane == r16, s, svec)
            z = svec * w + bb
            score_v[pl.ds(g * L, L)] = 1.0 / (1.0 + jnp.exp(-z))
            return 0
        lax.fori_loop(0, CHUNK // L, group_body, 0)

        pltpu.sync_copy(score_v, out.at[pl.ds(base, CHUNK)])
        return 0

    lax.fori_loop(0, n_chunks, chunk_body, 0)


def kernel(user_idx, item_idx, user_tables, item_tables, W, b):
    B, F = user_idx.shape
    _, V, D = user_tables.shape
    n_workers = 32
    rows_per_worker = B // n_workers
    n_chunks = rows_per_worker // CHUNK

    wb = jnp.zeros((L,), jnp.float32).at[0].set(W[0, 0]).at[1].set(b[0])

    mesh = plsc.VectorSubcoreMesh(core_axis_name="c", subcore_axis_name="s")
    fm = pl.kernel(
        functools.partial(_fm_body, F, V, D, rows_per_worker, n_chunks),
        out_type=jax.ShapeDtypeStruct((B,), jnp.float32),
        mesh=mesh,
        compiler_params=pltpu.CompilerParams(needs_layout_passes=False,
                                             use_tc_tiling_on_sc=False),
        scratch_types=[
            pltpu.VMEM((CHUNK, F), jnp.int32),
            pltpu.VMEM((CHUNK, F), jnp.int32),
            pltpu.VMEM((F, CHUNK), jnp.int32),
            pltpu.VMEM((F, CHUNK), jnp.int32),
            pltpu.VMEM((F, CHUNK, D), jnp.float32),
            pltpu.VMEM((F, CHUNK, D), jnp.float32),
            pltpu.VMEM((CHUNK,), jnp.float32),
            pltpu.VMEM((L,), jnp.float32),
            pltpu.SemaphoreType.DMA,
        ],
    )
    out = fm(user_tables, item_tables, user_idx, item_idx, wb)
    return out.reshape(B, 1)


# speedup vs baseline: 1.0892x; 1.0159x over previous
import functools

import jax
import jax.numpy as jnp
from jax import lax
from jax.experimental import pallas as pl
from jax.experimental.pallas import tpu as pltpu
from jax.experimental.pallas import tpu_sc as plsc

L = 16
CHUNK = 128


def _fm_body(F, V, D, rows_per_worker, n_chunks,
             utab, itab, uidx_t, iidx_t, wb_hbm, out,
             uidx_v, iidx_v, ubuf, ibuf, score_v, wb_v, sem):
    nc = 2
    wid = lax.axis_index("s") * nc + lax.axis_index("c")
    base0 = wid * rows_per_worker

    pltpu.sync_copy(wb_hbm, wb_v)
    wb = wb_v[:]
    w = wb[0]
    bb = wb[1]

    lane = lax.iota(jnp.int32, L)

    def chunk_body(c, _):
        base = base0 + c * CHUNK
        pltpu.sync_copy(uidx_t.at[:, pl.ds(base, CHUNK)], uidx_v)
        pltpu.sync_copy(iidx_t.at[:, pl.ds(base, CHUNK)], iidx_v)

        def off_body(j, _):
            for f in range(F):
                sl = (f, pl.ds(j * L, L))
                uidx_v[sl] = uidx_v[sl] + f * V
                iidx_v[sl] = iidx_v[sl] + f * V
            return 0
        lax.fori_loop(0, CHUNK // L, off_body, 0)

        copies = []
        for f in range(F):
            copies.append(pltpu.async_copy(utab.at[uidx_v.at[f]],
                                           ubuf.at[f], sem))
            copies.append(pltpu.async_copy(itab.at[iidx_v.at[f]],
                                           ibuf.at[f], sem))
        for cp in copies:
            cp.wait()

        def group_body(g, _):
            svec = jnp.zeros((L,), jnp.float32)
            for r16 in range(L):
                r = g * L + r16
                u = ubuf[0, r]
                v = ibuf[0, r]
                for f in range(1, F):
                    u = u + ubuf[f, r]
                    v = v + ibuf[f, r]
                s = jnp.sum(u * v)
                svec = jnp.where(lane == r16, s, svec)
            z = svec * w + bb
            score_v[pl.ds(g * L, L)] = 1.0 / (1.0 + jnp.exp(-z))
            return 0
        lax.fori_loop(0, CHUNK // L, group_body, 0)

        pltpu.sync_copy(score_v, out.at[pl.ds(base, CHUNK)])
        return 0

    lax.fori_loop(0, n_chunks, chunk_body, 0)


def kernel(user_idx, item_idx, user_tables, item_tables, W, b):
    B, F = user_idx.shape
    _, V, D = user_tables.shape
    n_workers = 32
    rows_per_worker = B // n_workers
    n_chunks = rows_per_worker // CHUNK

    utab = user_tables.reshape(F * V, D)
    itab = item_tables.reshape(F * V, D)
    uidx_t = user_idx.T
    iidx_t = item_idx.T
    wb = jnp.zeros((L,), jnp.float32).at[0].set(W[0, 0]).at[1].set(b[0])

    mesh = plsc.VectorSubcoreMesh(core_axis_name="c", subcore_axis_name="s")
    fm = pl.kernel(
        functools.partial(_fm_body, F, V, D, rows_per_worker, n_chunks),
        out_type=jax.ShapeDtypeStruct((B,), jnp.float32),
        mesh=mesh,
        compiler_params=pltpu.CompilerParams(needs_layout_passes=False,
                                             use_tc_tiling_on_sc=False),
        scratch_types=[
            pltpu.VMEM((F, CHUNK), jnp.int32),
            pltpu.VMEM((F, CHUNK), jnp.int32),
            pltpu.VMEM((F, CHUNK, D), jnp.float32),
            pltpu.VMEM((F, CHUNK, D), jnp.float32),
            pltpu.VMEM((CHUNK,), jnp.float32),
            pltpu.VMEM((L,), jnp.float32),
            pltpu.SemaphoreType.DMA,
        ],
    )
    out = fm(utab, itab, uidx_t, iidx_t, wb)
    return out.reshape(B, 1)
